# baseline (speedup 1.0000x reference)
import jax
import jax.numpy as jnp
from jax.experimental import pallas as pl
from jax.experimental.pallas import tpu as pltpu

_N = 10000
_D = 128
_B = 256
_W = 16
_NCHUNK = 4


def _row_gather_dma(idx_ref, src_ref, dst_ref, sem, nrows):
    def _copy(i):
        r = idx_ref[i]
        return pltpu.make_async_copy(
            src_ref.at[pl.ds(r, 1), :], dst_ref.at[pl.ds(i, 1), :], sem.at[i % _W]
        )

    def _prolog(i, c):
        _copy(i).start()
        return c

    jax.lax.fori_loop(0, _W, _prolog, 0)

    def _main(i, c):
        _copy(i).wait()

        @pl.when(i + _W < nrows)
        def _():
            _copy(i + _W).start()

        return c

    jax.lax.fori_loop(0, nrows, _main, 0)


def _attention_scores(S, a_v, b_row):
    colsum = jnp.sum(S, axis=0, keepdims=True)
    cnt = jnp.sum((colsum != 0.0).astype(jnp.int32))
    inv = 1.0 / cnt.astype(jnp.float32)
    att = a_v + b_row + 1.0
    att = inv * jax.nn.relu(att)
    p1 = jnp.sum(S * att, axis=0, keepdims=True)
    return p1, colsum


def _select_columns(S, p1, samp_row):
    csz = _N // _NCHUNK
    G = jnp.zeros((_B, _B), dtype=jnp.float32)
    pg = jnp.zeros((1, _B), dtype=jnp.float32)
    for c in range(_NCHUNK):
        iota = jax.lax.broadcasted_iota(jnp.int32, (csz, _B), 0) + c * csz
        oh = (iota == samp_row).astype(jnp.float32)
        dn = (((1,), (0,)), ((), ()))
        G = G + jax.lax.dot_general(
            S[:, c * csz:(c + 1) * csz], oh, dn,
            precision=jax.lax.Precision.HIGHEST)
        pg = pg + jax.lax.dot_general(
            p1[:, c * csz:(c + 1) * csz], oh, dn,
            precision=jax.lax.Precision.HIGHEST)
    return G * (1.0 / (pg * 256.0))


def _make_layer1():
    def body(v_ref, adj_ref, av_ref, brow_ref,
             S_ref, p1_ref, colsum_ref, sem):
        _row_gather_dma(v_ref, adj_ref, S_ref, sem, _B)
        p1, colsum = _attention_scores(S_ref[...], av_ref[...], brow_ref[...])
        p1_ref[...] = p1
        colsum_ref[...] = colsum

    return pl.pallas_call(
        body,
        in_specs=[
            pl.BlockSpec(memory_space=pltpu.SMEM),
            pl.BlockSpec(memory_space=pl.ANY),
            pl.BlockSpec(memory_space=pltpu.VMEM),
            pl.BlockSpec(memory_space=pltpu.VMEM),
        ],
        out_specs=[
            pl.BlockSpec(memory_space=pltpu.VMEM),
            pl.BlockSpec(memory_space=pltpu.VMEM),
            pl.BlockSpec(memory_space=pltpu.VMEM),
        ],
        out_shape=[
            jax.ShapeDtypeStruct((_B, _N), jnp.float32),
            jax.ShapeDtypeStruct((1, _N), jnp.float32),
            jax.ShapeDtypeStruct((1, _N), jnp.float32),
        ],
        scratch_shapes=[
            pltpu.SemaphoreType.DMA((_W,)),
        ],
    )


def _make_layer0():
    def body(cur_ref, adj_ref, av_ref, brow_ref, samp_row_ref,
             Sprev_ref, p1prev_ref,
             S_ref, p1_ref, colsum_ref, supprev_ref, sem):
        _row_gather_dma(cur_ref, adj_ref, S_ref, sem, _B)
        p1, colsum = _attention_scores(S_ref[...], av_ref[...], brow_ref[...])
        p1_ref[...] = p1
        colsum_ref[...] = colsum
        supprev_ref[...] = _select_columns(
            Sprev_ref[...], p1prev_ref[...], samp_row_ref[...])

    return pl.pallas_call(
        body,
        in_specs=[
            pl.BlockSpec(memory_space=pltpu.SMEM),
            pl.BlockSpec(memory_space=pl.ANY),
            pl.BlockSpec(memory_space=pltpu.VMEM),
            pl.BlockSpec(memory_space=pltpu.VMEM),
            pl.BlockSpec(memory_space=pltpu.VMEM),
            pl.BlockSpec(memory_space=pltpu.VMEM),
            pl.BlockSpec(memory_space=pltpu.VMEM),
        ],
        out_specs=[
            pl.BlockSpec(memory_space=pltpu.VMEM),
            pl.BlockSpec(memory_space=pltpu.VMEM),
            pl.BlockSpec(memory_space=pltpu.VMEM),
            pl.BlockSpec(memory_space=pltpu.VMEM),
        ],
        out_shape=[
            jax.ShapeDtypeStruct((_B, _N), jnp.float32),
            jax.ShapeDtypeStruct((1, _N), jnp.float32),
            jax.ShapeDtypeStruct((1, _N), jnp.float32),
            jax.ShapeDtypeStruct((_B, _B), jnp.float32),
        ],
        scratch_shapes=[
            pltpu.SemaphoreType.DMA((_W,)),
        ],
    )


def _make_loss():
    def body(samp0_ref, feat_ref, S0_ref, p10_ref, samp0_row_ref,
             mask_ref, pu_ref, X0_ref, sup0_ref, loss_ref):
        sup0_ref[...] = _select_columns(
            S0_ref[...], p10_ref[...], samp0_row_ref[...])

        def _x0(i, c):
            X0_ref[pl.ds(i, 1), :] = feat_ref[pl.ds(samp0_ref[i], 1), :]
            return c

        jax.lax.fori_loop(0, _B, _x0, 0)
        feat = feat_ref[...]
        means = jnp.sum(feat * mask_ref[...], axis=0, keepdims=True)
        fc = feat - means
        lv = jnp.sum(fc * fc * pu_ref[...], axis=0, keepdims=True)
        loss_ref[0, 0] = jnp.sum(lv) / jnp.float32(_D)

    return pl.pallas_call(
        body,
        in_specs=[
            pl.BlockSpec(memory_space=pltpu.SMEM),
            pl.BlockSpec(memory_space=pltpu.VMEM),
            pl.BlockSpec(memory_space=pltpu.VMEM),
            pl.BlockSpec(memory_space=pltpu.VMEM),
            pl.BlockSpec(memory_space=pltpu.VMEM),
            pl.BlockSpec(memory_space=pltpu.VMEM),
            pl.BlockSpec(memory_space=pltpu.VMEM),
        ],
        out_specs=[
            pl.BlockSpec(memory_space=pltpu.VMEM),
            pl.BlockSpec(memory_space=pltpu.VMEM),
            pl.BlockSpec(memory_space=pltpu.SMEM),
        ],
        out_shape=[
            jax.ShapeDtypeStruct((_B, _D), jnp.float32),
            jax.ShapeDtypeStruct((_B, _B), jnp.float32),
            jax.ShapeDtypeStruct((1, 1), jnp.float32),
        ],
    )


_layer1_call = _make_layer1()
_layer0_call = _make_layer0()
_loss_call = _make_loss()


def kernel(features, adj, w1, w2, v):
    key = jax.random.key(42)
    v32 = v.astype(jnp.int32)
    b_row = jnp.matmul(features, w2).reshape(1, -1)

    av1 = jnp.matmul(features[v32], w1)
    S1, p1r_1, colsum1 = _layer1_call(v32, adj, av1, b_row)
    p1_1 = p1r_1.reshape(-1)
    p_1 = p1_1 / jnp.sum(p1_1)
    sampled1 = jax.random.choice(
        jax.random.fold_in(key, 1), adj.shape[1], shape=(_B,), replace=True, p=p_1)
    s1 = sampled1.astype(jnp.int32)

    av0 = jnp.matmul(features[s1], w1)
    S0, p1r_0, colsum0, support1 = _layer0_call(
        s1, adj, av0, b_row, s1.reshape(1, _B), S1, p1r_1)
    p1_0 = p1r_0.reshape(-1)
    p_0 = p1_0 / jnp.sum(p1_0)
    sampled0 = jax.random.choice(
        jax.random.fold_in(key, 0), adj.shape[1], shape=(_B,), replace=True, p=p_0)
    s0 = sampled0.astype(jnp.int32)

    mask = (colsum1.reshape(-1) != 0.0).astype(jnp.float32).reshape(_N, 1)
    p_u = p_1.reshape(_N, 1)
    X0, support0, loss = _loss_call(
        s0, features, S0, p1r_0, s0.reshape(1, _B), mask, p_u)
    return (X0, support0, support1, loss.reshape(()))


# speedup vs baseline: 1.5059x; 1.5059x over previous
import jax
import jax.numpy as jnp
from jax.experimental import pallas as pl
from jax.experimental.pallas import tpu as pltpu

_N = 10000
_D = 128
_B = 256
_W = 16
_NCHUNK = 4


def _row_gather_dma(idx_ref, src_ref, dst_ref, sem, nrows):
    def _copy(i):
        r = idx_ref[i, 0]
        return pltpu.make_async_copy(
            src_ref.at[pl.ds(r, 1), :], dst_ref.at[pl.ds(i, 1), :], sem.at[i % _W]
        )

    def _prolog(i, c):
        _copy(i).start()
        return c

    jax.lax.fori_loop(0, _W, _prolog, 0)

    def _main(i, c):
        _copy(i).wait()

        @pl.when(i + _W < nrows)
        def _():
            _copy(i + _W).start()

        return c

    jax.lax.fori_loop(0, nrows, _main, 0)


def _attention_scores(S, a_v, b_row):
    colsum = jnp.sum(S, axis=0, keepdims=True)
    cnt = jnp.sum((colsum != 0.0).astype(jnp.int32))
    inv = 1.0 / cnt.astype(jnp.float32)
    att = a_v + b_row + 1.0
    att = inv * jax.nn.relu(att)
    p1 = jnp.sum(S * att, axis=0, keepdims=True)
    return p1, colsum


def _cumsum_rw(p_row):
    pp = jnp.concatenate(
        [p_row, jnp.zeros((1, 79 * 128 - _N), jnp.float32)], axis=1)
    X = pp.reshape(79, 128)
    for d in (1, 2, 4, 8, 16, 32, 64):
        X = X + jnp.concatenate(
            [jnp.zeros((79, d), jnp.float32), X[:, :128 - d]], axis=1)
    rows = X[:, 127:128]
    Y = jnp.concatenate([jnp.zeros((1, 1), jnp.float32), rows], axis=0)
    for d in (1, 2, 4, 8, 16, 32, 64):
        Y = Y + jnp.concatenate(
            [jnp.zeros((d, 1), jnp.float32), Y[:80 - d, :]], axis=0)
    offs = Y[:79, :]
    C = X + offs
    return C.reshape(1, 79 * 128)[:, :_N]


def _sample(p1, u_col):
    p = p1 / jnp.sum(p1)
    cum = _cumsum_rw(p)
    last = cum[:, _N - 1:_N]
    r = last * (1.0 - u_col)
    mask = (cum < r).astype(jnp.int32)
    ind = jnp.sum(mask, axis=1, keepdims=True)
    return ind, p


def _select_columns(S, p1, samp_row):
    csz = _N // _NCHUNK
    G = jnp.zeros((_B, _B), dtype=jnp.float32)
    pg = jnp.zeros((1, _B), dtype=jnp.float32)
    for c in range(_NCHUNK):
        iota = jax.lax.broadcasted_iota(jnp.int32, (csz, _B), 0) + c * csz
        oh = (iota == samp_row).astype(jnp.float32)
        dn = (((1,), (0,)), ((), ()))
        G = G + jax.lax.dot_general(
            S[:, c * csz:(c + 1) * csz], oh, dn,
            precision=jax.lax.Precision.HIGHEST)
        pg = pg + jax.lax.dot_general(
            p1[:, c * csz:(c + 1) * csz], oh, dn,
            precision=jax.lax.Precision.HIGHEST)
    return G * (1.0 / (pg * 256.0))


def _make_fused():
    def body(v_ref, adj_ref, feat_ref, av1_ref, cv_ref, brow_ref,
             u1_ref, u0_ref,
             X0_ref, sup0_ref, sup1_ref, loss_ref,
             S1_ref, S0_ref, av0_ref, ivm_ref, ism_ref, sem, isem):
        _row_gather_dma(v_ref, adj_ref, S1_ref, sem, _B)
        p1_1, colsum1 = _attention_scores(S1_ref[...], av1_ref[...], brow_ref[...])
        s1, p_u = _sample(p1_1, u1_ref[...])

        ivm_ref[...] = s1
        cp = pltpu.make_async_copy(ivm_ref, ism_ref, isem)
        cp.start()
        cp.wait()

        _row_gather_dma(ism_ref, adj_ref, S0_ref, sem, _B)

        def _av0(i, c):
            av0_ref[pl.ds(i, 1), :] = cv_ref[pl.ds(ism_ref[i, 0], 1), :]
            return c

        jax.lax.fori_loop(0, _B, _av0, 0)
        p1_0, colsum0 = _attention_scores(S0_ref[...], av0_ref[...], brow_ref[...])

        sup1_ref[...] = _select_columns(S1_ref[...], p1_1, s1.reshape(1, _B))

        s0, _ = _sample(p1_0, u0_ref[...])
        ivm_ref[...] = s0
        cp = pltpu.make_async_copy(ivm_ref, ism_ref, isem)
        cp.start()
        cp.wait()

        sup0_ref[...] = _select_columns(S0_ref[...], p1_0, s0.reshape(1, _B))

        def _x0(i, c):
            X0_ref[pl.ds(i, 1), :] = feat_ref[pl.ds(ism_ref[i, 0], 1), :]
            return c

        jax.lax.fori_loop(0, _B, _x0, 0)

        feat = feat_ref[...]
        m = (colsum1 != 0.0).astype(jnp.float32).reshape(_N, 1)
        pu_col = p_u.reshape(_N, 1)
        means = jnp.sum(feat * m, axis=0, keepdims=True)
        fc = feat - means
        lv = jnp.sum(fc * fc * pu_col, axis=0, keepdims=True)
        loss_ref[0, 0] = jnp.sum(lv) / jnp.float32(_D)

    return pl.pallas_call(
        body,
        in_specs=[
            pl.BlockSpec(memory_space=pltpu.SMEM),
            pl.BlockSpec(memory_space=pl.ANY),
            pl.BlockSpec(memory_space=pltpu.VMEM),
            pl.BlockSpec(memory_space=pltpu.VMEM),
            pl.BlockSpec(memory_space=pltpu.VMEM),
            pl.BlockSpec(memory_space=pltpu.VMEM),
            pl.BlockSpec(memory_space=pltpu.VMEM),
            pl.BlockSpec(memory_space=pltpu.VMEM),
        ],
        out_specs=[
            pl.BlockSpec(memory_space=pltpu.VMEM),
            pl.BlockSpec(memory_space=pltpu.VMEM),
            pl.BlockSpec(memory_space=pltpu.VMEM),
            pl.BlockSpec(memory_space=pltpu.SMEM),
        ],
        out_shape=[
            jax.ShapeDtypeStruct((_B, _D), jnp.float32),
            jax.ShapeDtypeStruct((_B, _B), jnp.float32),
            jax.ShapeDtypeStruct((_B, _B), jnp.float32),
            jax.ShapeDtypeStruct((1, 1), jnp.float32),
        ],
        scratch_shapes=[
            pltpu.VMEM((_B, _N), jnp.float32),
            pltpu.VMEM((_B, _N), jnp.float32),
            pltpu.VMEM((_B, 1), jnp.float32),
            pltpu.VMEM((_B, 1), jnp.int32),
            pltpu.SMEM((_B, 1), jnp.int32),
            pltpu.SemaphoreType.DMA((_W,)),
            pltpu.SemaphoreType.DMA,
        ],
        compiler_params=pltpu.CompilerParams(
            vmem_limit_bytes=100 * 1024 * 1024,
        ),
    )


_fused_call = _make_fused()


def kernel(features, adj, w1, w2, v):
    key = jax.random.key(42)
    v32 = v.astype(jnp.int32)
    b_row = jnp.matmul(features, w2).reshape(1, -1)
    av1 = jnp.matmul(features[v32], w1)
    c_v = jnp.matmul(features, w1)
    u1 = jax.random.uniform(
        jax.random.fold_in(key, 1), (_B,), dtype=jnp.float32).reshape(_B, 1)
    u0 = jax.random.uniform(
        jax.random.fold_in(key, 0), (_B,), dtype=jnp.float32).reshape(_B, 1)

    X0, support0, support1, loss = _fused_call(
        v32.reshape(_B, 1), adj, features, av1, c_v, b_row, u1, u0)
    return (X0, support0, support1, loss.reshape(()))


# speedup vs baseline: 1.7663x; 1.1730x over previous
import jax
import jax.numpy as jnp
from jax.experimental import pallas as pl
from jax.experimental.pallas import tpu as pltpu

_N = 10000
_D = 128
_B = 256
_W = 64
_NCHUNK = 4


def _row_gather_dma(idx_ref, src_ref, dst_ref, sem, nrows):
    def _copy(i):
        r = idx_ref[i, 0]
        return pltpu.make_async_copy(
            src_ref.at[pl.ds(r, 1), :], dst_ref.at[pl.ds(i, 1), :], sem.at[i % _W]
        )

    def _prolog(i, c):
        _copy(i).start()
        return c

    jax.lax.fori_loop(0, _W, _prolog, 0)

    def _main(i, c):
        _copy(i).wait()

        @pl.when(i + _W < nrows)
        def _():
            _copy(i + _W).start()

        return c

    jax.lax.fori_loop(0, nrows, _main, 0)


def _attention_scores(S, a_v, b_row):
    colsum = jnp.sum(S, axis=0, keepdims=True)
    cnt = jnp.sum((colsum != 0.0).astype(jnp.int32))
    inv = 1.0 / cnt.astype(jnp.float32)
    att = a_v + b_row + 1.0
    att = inv * jax.nn.relu(att)
    p1 = jnp.sum(S * att, axis=0, keepdims=True)
    return p1, colsum


def _cumsum_rw(p_row):
    pp = jnp.concatenate(
        [p_row, jnp.zeros((1, 79 * 128 - _N), jnp.float32)], axis=1)
    X = pp.reshape(79, 128)
    for d in (1, 2, 4, 8, 16, 32, 64):
        X = X + jnp.concatenate(
            [jnp.zeros((79, d), jnp.float32), X[:, :128 - d]], axis=1)
    rows = X[:, 127:128]
    Y = jnp.concatenate([jnp.zeros((1, 1), jnp.float32), rows], axis=0)
    for d in (1, 2, 4, 8, 16, 32, 64):
        Y = Y + jnp.concatenate(
            [jnp.zeros((d, 1), jnp.float32), Y[:80 - d, :]], axis=0)
    offs = Y[:79, :]
    C = X + offs
    return C.reshape(1, 79 * 128)[:, :_N]


def _sample(p1, u_col):
    p = p1 / jnp.sum(p1)
    cum = _cumsum_rw(p)
    last = cum[:, _N - 1:_N]
    r = last * (1.0 - u_col)
    mask = (cum < r).astype(jnp.int32)
    ind = jnp.sum(mask, axis=1, keepdims=True)
    return ind, p


def _select_columns(S, p1, samp_row):
    csz = _N // _NCHUNK
    G = jnp.zeros((_B, _B), dtype=jnp.float32)
    pg = jnp.zeros((1, _B), dtype=jnp.float32)
    for c in range(_NCHUNK):
        iota = jax.lax.broadcasted_iota(jnp.int32, (csz, _B), 0) + c * csz
        oh = (iota == samp_row).astype(jnp.float32)
        dn = (((1,), (0,)), ((), ()))
        G = G + jax.lax.dot_general(
            S[:, c * csz:(c + 1) * csz], oh, dn,
            precision=jax.lax.Precision.HIGHEST)
        pg = pg + jax.lax.dot_general(
            p1[:, c * csz:(c + 1) * csz], oh, dn,
            precision=jax.lax.Precision.HIGHEST)
    return G * (1.0 / (pg * 256.0))


def _make_fused():
    def body(v_ref, adj_ref, feat_ref, av1_ref, cv_ref, brow_ref,
             u1_ref, u0_ref,
             X0_ref, sup0_ref, sup1_ref, loss_ref,
             S1_ref, S0_ref, av0_ref, ivm_ref, ism_ref, sem, isem):
        _row_gather_dma(v_ref, adj_ref, S1_ref, sem, _B)
        p1_1, colsum1 = _attention_scores(S1_ref[...], av1_ref[...], brow_ref[...])
        s1, p_u = _sample(p1_1, u1_ref[...])

        ivm_ref[...] = s1
        cp = pltpu.make_async_copy(ivm_ref, ism_ref, isem)
        cp.start()
        cp.wait()

        _row_gather_dma(ism_ref, adj_ref, S0_ref, sem, _B)

        def _av0(i, c):
            av0_ref[pl.ds(i, 1), :] = cv_ref[pl.ds(ism_ref[i, 0], 1), :]
            return c

        jax.lax.fori_loop(0, _B, _av0, 0)
        p1_0, colsum0 = _attention_scores(S0_ref[...], av0_ref[...], brow_ref[...])

        sup1_ref[...] = _select_columns(S1_ref[...], p1_1, s1.reshape(1, _B))

        s0, _ = _sample(p1_0, u0_ref[...])
        ivm_ref[...] = s0
        cp = pltpu.make_async_copy(ivm_ref, ism_ref, isem)
        cp.start()
        cp.wait()

        sup0_ref[...] = _select_columns(S0_ref[...], p1_0, s0.reshape(1, _B))

        def _x0(i, c):
            X0_ref[pl.ds(i, 1), :] = feat_ref[pl.ds(ism_ref[i, 0], 1), :]
            return c

        jax.lax.fori_loop(0, _B, _x0, 0)

        feat = feat_ref[...]
        m = (colsum1 != 0.0).astype(jnp.float32).reshape(_N, 1)
        pu_col = p_u.reshape(_N, 1)
        means = jnp.sum(feat * m, axis=0, keepdims=True)
        fc = feat - means
        lv = jnp.sum(fc * fc * pu_col, axis=0, keepdims=True)
        loss_ref[0, 0] = jnp.sum(lv) / jnp.float32(_D)

    return pl.pallas_call(
        body,
        in_specs=[
            pl.BlockSpec(memory_space=pltpu.SMEM),
            pl.BlockSpec(memory_space=pl.ANY),
            pl.BlockSpec(memory_space=pltpu.VMEM),
            pl.BlockSpec(memory_space=pltpu.VMEM),
            pl.BlockSpec(memory_space=pltpu.VMEM),
            pl.BlockSpec(memory_space=pltpu.VMEM),
            pl.BlockSpec(memory_space=pltpu.VMEM),
            pl.BlockSpec(memory_space=pltpu.VMEM),
        ],
        out_specs=[
            pl.BlockSpec(memory_space=pltpu.VMEM),
            pl.BlockSpec(memory_space=pltpu.VMEM),
            pl.BlockSpec(memory_space=pltpu.VMEM),
            pl.BlockSpec(memory_space=pltpu.SMEM),
        ],
        out_shape=[
            jax.ShapeDtypeStruct((_B, _D), jnp.float32),
            jax.ShapeDtypeStruct((_B, _B), jnp.float32),
            jax.ShapeDtypeStruct((_B, _B), jnp.float32),
            jax.ShapeDtypeStruct((1, 1), jnp.float32),
        ],
        scratch_shapes=[
            pltpu.VMEM((_B, _N), jnp.float32),
            pltpu.VMEM((_B, _N), jnp.float32),
            pltpu.VMEM((_B, 1), jnp.float32),
            pltpu.VMEM((_B, 1), jnp.int32),
            pltpu.SMEM((_B, 1), jnp.int32),
            pltpu.SemaphoreType.DMA((_W,)),
            pltpu.SemaphoreType.DMA,
        ],
        compiler_params=pltpu.CompilerParams(
            vmem_limit_bytes=100 * 1024 * 1024,
        ),
    )


_fused_call = _make_fused()


def kernel(features, adj, w1, w2, v):
    key = jax.random.key(42)
    v32 = v.astype(jnp.int32)
    b_row = jnp.matmul(features, w2).reshape(1, -1)
    av1 = jnp.matmul(features[v32], w1)
    c_v = jnp.matmul(features, w1)
    u1 = jax.random.uniform(
        jax.random.fold_in(key, 1), (_B,), dtype=jnp.float32).reshape(_B, 1)
    u0 = jax.random.uniform(
        jax.random.fold_in(key, 0), (_B,), dtype=jnp.float32).reshape(_B, 1)

    X0, support0, support1, loss = _fused_call(
        v32.reshape(_B, 1), adj, features, av1, c_v, b_row, u1, u0)
    return (X0, support0, support1, loss.reshape(()))
